# baseline (speedup 1.0000x reference)
import math

import jax
import jax.numpy as jnp
from jax import lax
from jax.experimental import pallas as pl
from jax.experimental.pallas import tpu as pltpu
from jax.experimental.pallas import tpu_sc as plsc

VOCAB = 1000000
D = 64
NB = 16384
NT = 50
B_TOTAL = NB * NT
NC, NS = 2, 16
NW = NC * NS
GRP = 128
KB = 4
SB_ROWS = KB * GRP
N_BLK_B = NB // GRP
SB_PER_T = N_BLK_B // KB
N_SB = NT * SB_PER_T
SB_PER_W = N_SB // NW
FR = D // 8
SCALE = math.sqrt(D)


def _emb_kernel(w_hbm, idx_hbm, out_hbm, idx_v, rows, trans, gs0, gs1, osem):
    wid = lax.axis_index("s") * NC + lax.axis_index("c")
    gs_base = wid * SB_PER_W
    gsems = (gs0, gs1)
    iota = lax.iota(jnp.int32, 16)

    def stage_idx(s, b):
        roff = pl.multiple_of((gs_base + s) * KB, 4)
        pltpu.sync_copy(idx_hbm.at[pl.ds(roff, KB)], idx_v.at[b])

    def fire_gather(b):
        for k in range(KB):
            pltpu.async_copy(
                w_hbm.at[idx_v.at[b, k]],
                rows.at[b, pl.ds(k * GRP, GRP)],
                gsems[b],
            )

    def wait_gather(b):
        pltpu.make_async_copy(
            w_hbm.at[pl.ds(0, SB_ROWS)], rows.at[b], gsems[b]
        ).wait()

    def transpose_scale(b):
        @pl.loop(0, FR)
        def _(fr):
            for bci in range(KB):
                for fi in range(8):
                    cvec = jnp.full((16,), fr * 8 + fi, jnp.int32)
                    for j in range(GRP // 16):
                        ridx = iota + (bci * GRP + j * 16)
                        v = plsc.load_gather(rows.at[b], [ridx, cvec])
                        trans[fr, bci, fi, pl.ds(j * 16, 16)] = v * SCALE

    def fire_out(s, b):
        gs = gs_base + s
        t = lax.shift_right_logical(gs, 5)
        bq = lax.bitwise_and(gs, SB_PER_T - 1)
        bc0 = pl.multiple_of(bq * KB, 4)
        for fr in range(FR):
            pltpu.async_copy(
                trans.at[fr],
                out_hbm.at[t, fr, pl.ds(bc0, KB)],
                osem,
            )

    def wait_out():
        pltpu.make_async_copy(
            w_hbm.at[pl.ds(0, SB_ROWS)], trans, osem
        ).wait()

    stage_idx(0, 0)
    fire_gather(0)

    stage_idx(1, 1)
    fire_gather(1)
    wait_gather(0)
    transpose_scale(0)
    fire_out(0, 0)

    stage_idx(2, 0)
    fire_gather(0)
    wait_gather(1)
    wait_out()
    transpose_scale(1)
    fire_out(1, 1)

    @pl.loop(2, SB_PER_W - 2, step=2)
    def _(s0):
        for u in range(2):
            s = s0 + u
            b = u
            stage_idx(s + 1, 1 - b)
            fire_gather(1 - b)
            wait_gather(b)
            wait_out()
            transpose_scale(b)
            fire_out(s, b)

    s2 = SB_PER_W - 2
    stage_idx(s2 + 1, 1)
    fire_gather(1)
    wait_gather(0)
    wait_out()
    transpose_scale(0)
    fire_out(s2, 0)

    wait_gather(1)
    wait_out()
    transpose_scale(1)
    fire_out(s2 + 1, 1)
    wait_out()


@jax.jit
def _emb(weight, idx2d):
    mesh = plsc.VectorSubcoreMesh(
        core_axis_name="c", subcore_axis_name="s", num_cores=NC, num_subcores=NS
    )
    run = pl.kernel(
        _emb_kernel,
        out_type=jax.ShapeDtypeStruct((NT, FR, N_BLK_B, 8, GRP), jnp.float32),
        mesh=mesh,
        scratch_types=[
            pltpu.VMEM((2, KB, GRP), jnp.int32),
            pltpu.VMEM((2, SB_ROWS, D), jnp.float32),
            pltpu.VMEM((FR, KB, 8, GRP), jnp.float32),
            pltpu.SemaphoreType.DMA,
            pltpu.SemaphoreType.DMA,
            pltpu.SemaphoreType.DMA,
        ],
        compiler_params=pltpu.CompilerParams(
            use_tc_tiling_on_sc=False, needs_layout_passes=False
        ),
    )
    return run(weight, idx2d)


def kernel(input_ids, weight):
    idx2d = input_ids.astype(jnp.int32).T.reshape(B_TOTAL // GRP, GRP)
    raw5 = _emb(weight, idx2d)
    out = raw5.transpose(2, 4, 0, 1, 3).reshape(NB, NT, D)
    return out


# speedup vs baseline: 1.7553x; 1.7553x over previous
import math

import jax
import jax.numpy as jnp
from jax import lax
from jax.experimental import pallas as pl
from jax.experimental.pallas import tpu as pltpu
from jax.experimental.pallas import tpu_sc as plsc

VOCAB = 1000000
D = 64
NB = 16384
NT = 50
B_TOTAL = NB * NT
NC, NS = 2, 16
NW = NC * NS
GRP = 128
KB = 4
SB_ROWS = KB * GRP
N_BLK_B = NB // GRP
SB_PER_T = N_BLK_B // KB
N_SB = NT * SB_PER_T
SB_PER_W = N_SB // NW
FR = D // 8
SCALE = math.sqrt(D)


def _emb_kernel(w_hbm, idx_hbm, out_hbm, idx_v, rows, trans, gs0, gs1, osem):
    wid = lax.axis_index("s") * NC + lax.axis_index("c")
    gs_base = wid * SB_PER_W
    gsems = (gs0, gs1)
    iota = lax.iota(jnp.int32, 16)

    def stage_idx(s, b):
        roff = pl.multiple_of((gs_base + s) * KB, 4)
        pltpu.sync_copy(idx_hbm.at[pl.ds(roff, KB)], idx_v.at[b])

    def fire_gather(b):
        for k in range(KB):
            pltpu.async_copy(
                w_hbm.at[idx_v.at[b, k]],
                rows.at[b, pl.ds(k * GRP, GRP)],
                gsems[b],
            )

    def wait_gather(b):
        pltpu.make_async_copy(
            w_hbm.at[pl.ds(0, SB_ROWS)], rows.at[b], gsems[b]
        ).wait()

    fi_iota = lax.rem(iota, 8)
    fr_half = lax.div(iota, 8)

    def transpose_scale(b):
        @pl.loop(0, GRP)
        def _(bi):
            bi_v = jnp.full((16,), 0, jnp.int32) + bi
            for bci in range(KB):
                bci_v = jnp.full((16,), bci, jnp.int32)
                for j in range(D // 16):
                    fr_v = fr_half + (2 * j)
                    v = rows[b, bci * GRP + bi, pl.ds(j * 16, 16)]
                    plsc.store_scatter(
                        trans, [fr_v, bci_v, fi_iota, bi_v], v * SCALE
                    )

    def fire_out(s, b):
        gs = gs_base + s
        t = lax.shift_right_logical(gs, 5)
        bq = lax.bitwise_and(gs, SB_PER_T - 1)
        bc0 = pl.multiple_of(bq * KB, 4)
        for fr in range(FR):
            pltpu.async_copy(
                trans.at[fr, :, :, pl.ds(0, GRP)],
                out_hbm.at[t, fr, pl.ds(bc0, KB)],
                osem,
            )

    def wait_out():
        pltpu.make_async_copy(
            w_hbm.at[pl.ds(0, SB_ROWS)], rows.at[0], osem
        ).wait()

    stage_idx(0, 0)
    fire_gather(0)

    stage_idx(1, 1)
    fire_gather(1)
    wait_gather(0)
    transpose_scale(0)
    fire_out(0, 0)

    stage_idx(2, 0)
    fire_gather(0)
    wait_gather(1)
    wait_out()
    transpose_scale(1)
    fire_out(1, 1)

    @pl.loop(2, SB_PER_W - 2, step=2)
    def _(s0):
        for u in range(2):
            s = s0 + u
            b = u
            stage_idx(s + 1, 1 - b)
            fire_gather(1 - b)
            wait_gather(b)
            wait_out()
            transpose_scale(b)
            fire_out(s, b)

    s2 = SB_PER_W - 2
    stage_idx(s2 + 1, 1)
    fire_gather(1)
    wait_gather(0)
    wait_out()
    transpose_scale(0)
    fire_out(s2, 0)

    wait_gather(1)
    wait_out()
    transpose_scale(1)
    fire_out(s2 + 1, 1)
    wait_out()


@jax.jit
def _emb(weight, idx2d):
    mesh = plsc.VectorSubcoreMesh(
        core_axis_name="c", subcore_axis_name="s", num_cores=NC, num_subcores=NS
    )
    run = pl.kernel(
        _emb_kernel,
        out_type=jax.ShapeDtypeStruct((NT, FR, N_BLK_B, 8, GRP), jnp.float32),
        mesh=mesh,
        scratch_types=[
            pltpu.VMEM((2, KB, GRP), jnp.int32),
            pltpu.VMEM((2, SB_ROWS, D), jnp.float32),
            pltpu.VMEM((FR, KB, 8, GRP + 1), jnp.float32),
            pltpu.SemaphoreType.DMA,
            pltpu.SemaphoreType.DMA,
            pltpu.SemaphoreType.DMA,
        ],
        compiler_params=pltpu.CompilerParams(
            use_tc_tiling_on_sc=False, needs_layout_passes=False
        ),
    )
    return run(weight, idx2d)


def kernel(input_ids, weight):
    idx2d = input_ids.astype(jnp.int32).T.reshape(B_TOTAL // GRP, GRP)
    raw5 = _emb(weight, idx2d)
    out = raw5.transpose(2, 4, 0, 1, 3).reshape(NB, NT, D)
    return out


# speedup vs baseline: 1.8086x; 1.0304x over previous
import math

import jax
import jax.numpy as jnp
from jax import lax
from jax.experimental import pallas as pl
from jax.experimental.pallas import tpu as pltpu
from jax.experimental.pallas import tpu_sc as plsc

VOCAB = 1000000
D = 64
NB = 16384
NT = 50
B_TOTAL = NB * NT
NC, NS = 2, 16
NW = NC * NS
GRP = 128
KB = 4
SB_ROWS = KB * GRP
N_BLK_B = NB // GRP
SB_PER_T = N_BLK_B // KB
N_SB = NT * SB_PER_T
SB_PER_W = N_SB // NW
FR = D // 8
SCALE = math.sqrt(D)


def _emb_kernel(w_hbm, idx_hbm, out_hbm, idx_v, rows, trans, gs0, gs1, osem, isem):
    wid = lax.axis_index("s") * NC + lax.axis_index("c")
    gs_base = wid * SB_PER_W
    gsems = (gs0, gs1)
    iota = lax.iota(jnp.int32, 16)

    def stage_idx(s, b):
        roff = pl.multiple_of((gs_base + s) * KB, 4)
        pltpu.async_copy(idx_hbm.at[pl.ds(roff, KB)], idx_v.at[b], isem)

    def wait_idx(b):
        pltpu.make_async_copy(
            idx_hbm.at[pl.ds(0, KB)], idx_v.at[b], isem
        ).wait()

    def fire_gather(b):
        for k in range(KB):
            pltpu.async_copy(
                w_hbm.at[idx_v.at[b, k]],
                rows.at[b, pl.ds(k * GRP, GRP)],
                gsems[b],
            )

    def wait_gather(b):
        pltpu.make_async_copy(
            w_hbm.at[pl.ds(0, SB_ROWS)], rows.at[b], gsems[b]
        ).wait()

    fi_iota = lax.rem(iota, 8)
    fr_half = lax.div(iota, 8)

    def transpose_scale(b):
        @pl.loop(0, GRP)
        def _(bi):
            bi_v = jnp.full((16,), 0, jnp.int32) + bi
            for bci in range(KB):
                bci_v = jnp.full((16,), bci, jnp.int32)
                for j in range(D // 16):
                    fr_v = fr_half + (2 * j)
                    v = rows[b, bci * GRP + bi, pl.ds(j * 16, 16)]
                    plsc.store_scatter(
                        trans, [fr_v, bci_v, fi_iota, bi_v], v * SCALE
                    )

    def fire_out(s, b):
        gs = gs_base + s
        t = lax.shift_right_logical(gs, 5)
        bq = lax.bitwise_and(gs, SB_PER_T - 1)
        bc0 = pl.multiple_of(bq * KB, 4)
        for fr in range(FR):
            pltpu.async_copy(
                trans.at[fr, :, :, pl.ds(0, GRP)],
                out_hbm.at[t, fr, pl.ds(bc0, KB)],
                osem,
            )

    def wait_out():
        pltpu.make_async_copy(
            w_hbm.at[pl.ds(0, SB_ROWS)], rows.at[0], osem
        ).wait()

    stage_idx(0, 0)
    wait_idx(0)
    fire_gather(0)
    stage_idx(1, 1)

    wait_idx(1)
    fire_gather(1)
    wait_gather(0)
    stage_idx(2, 0)
    transpose_scale(0)
    fire_out(0, 0)

    wait_idx(0)
    fire_gather(0)
    wait_gather(1)
    stage_idx(3, 1)
    wait_out()
    transpose_scale(1)
    fire_out(1, 1)

    @pl.loop(2, SB_PER_W - 2, step=2)
    def _(s0):
        for u in range(2):
            s = s0 + u
            b = u
            wait_idx(1 - b)
            fire_gather(1 - b)
            wait_gather(b)
            stage_idx(s + 2, b)
            wait_out()
            transpose_scale(b)
            fire_out(s, b)

    s2 = SB_PER_W - 2
    wait_idx(1)
    fire_gather(1)
    wait_gather(0)
    wait_out()
    transpose_scale(0)
    fire_out(s2, 0)

    wait_gather(1)
    wait_out()
    transpose_scale(1)
    fire_out(s2 + 1, 1)
    wait_out()


@jax.jit
def _emb(weight, idx2d):
    mesh = plsc.VectorSubcoreMesh(
        core_axis_name="c", subcore_axis_name="s", num_cores=NC, num_subcores=NS
    )
    run = pl.kernel(
        _emb_kernel,
        out_type=jax.ShapeDtypeStruct((NT, FR, N_BLK_B, 8, GRP), jnp.float32),
        mesh=mesh,
        scratch_types=[
            pltpu.VMEM((2, KB, GRP), jnp.int32),
            pltpu.VMEM((2, SB_ROWS, D), jnp.float32),
            pltpu.VMEM((FR, KB, 8, GRP + 1), jnp.float32),
            pltpu.SemaphoreType.DMA,
            pltpu.SemaphoreType.DMA,
            pltpu.SemaphoreType.DMA,
            pltpu.SemaphoreType.DMA,
        ],
        compiler_params=pltpu.CompilerParams(
            use_tc_tiling_on_sc=False, needs_layout_passes=False
        ),
    )
    return run(weight, idx2d)


def kernel(input_ids, weight):
    idx2d = input_ids.astype(jnp.int32).T.reshape(B_TOTAL // GRP, GRP)
    raw5 = _emb(weight, idx2d)
    out = raw5.transpose(2, 4, 0, 1, 3).reshape(NB, NT, D)
    return out


# speedup vs baseline: 1.8089x; 1.0001x over previous
import math

import jax
import jax.numpy as jnp
from jax import lax
from jax.experimental import pallas as pl
from jax.experimental.pallas import tpu as pltpu
from jax.experimental.pallas import tpu_sc as plsc

VOCAB = 1000000
D = 64
NB = 16384
NT = 50
B_TOTAL = NB * NT
NC, NS = 2, 16
NW = NC * NS
GRP = 128
KB = 4
SB_ROWS = KB * GRP
DESC = 256
NDESC = SB_ROWS // DESC
N_BLK_B = NB // GRP
SB_PER_T = N_BLK_B // KB
N_SB = NT * SB_PER_T
SB_PER_W = N_SB // NW
FR = D // 8
SCALE = math.sqrt(D)


def _emb_kernel(w_hbm, idx_hbm, out_hbm, idx_v, rows, trans, gs0, gs1, osem, isem):
    wid = lax.axis_index("s") * NC + lax.axis_index("c")
    gs_base = wid * SB_PER_W
    gsems = (gs0, gs1)
    iota = lax.iota(jnp.int32, 16)

    def stage_idx(s, b):
        roff = pl.multiple_of((gs_base + s) * NDESC, 2)
        pltpu.async_copy(idx_hbm.at[pl.ds(roff, NDESC)], idx_v.at[b], isem)

    def wait_idx(b):
        pltpu.make_async_copy(
            idx_hbm.at[pl.ds(0, NDESC)], idx_v.at[b], isem
        ).wait()

    def fire_gather(b):
        for k in range(NDESC):
            pltpu.async_copy(
                w_hbm.at[idx_v.at[b, k]],
                rows.at[b, pl.ds(k * DESC, DESC)],
                gsems[b],
            )

    def wait_gather(b):
        pltpu.make_async_copy(
            w_hbm.at[pl.ds(0, SB_ROWS)], rows.at[b], gsems[b]
        ).wait()

    fi_iota = lax.rem(iota, 8)
    fr_half = lax.div(iota, 8)

    def transpose_scale(b):
        @pl.loop(0, GRP)
        def _(bi):
            bi_v = jnp.full((16,), 0, jnp.int32) + bi
            for bci in range(KB):
                bci_v = jnp.full((16,), bci, jnp.int32)
                for j in range(D // 16):
                    fr_v = fr_half + (2 * j)
                    v = rows[b, bci * GRP + bi, pl.ds(j * 16, 16)]
                    plsc.store_scatter(
                        trans, [fr_v, bci_v, fi_iota, bi_v], v * SCALE
                    )

    def fire_out(s, b):
        gs = gs_base + s
        t = lax.shift_right_logical(gs, 5)
        bq = lax.bitwise_and(gs, SB_PER_T - 1)
        bc0 = pl.multiple_of(bq * KB, 4)
        for fr in range(FR):
            pltpu.async_copy(
                trans.at[fr, :, :, pl.ds(0, GRP)],
                out_hbm.at[t, fr, pl.ds(bc0, KB)],
                osem,
            )

    def wait_out():
        pltpu.make_async_copy(
            w_hbm.at[pl.ds(0, SB_ROWS)], rows.at[0], osem
        ).wait()

    stage_idx(0, 0)
    wait_idx(0)
    fire_gather(0)
    stage_idx(1, 1)

    wait_idx(1)
    fire_gather(1)
    wait_gather(0)
    stage_idx(2, 0)
    transpose_scale(0)
    fire_out(0, 0)

    wait_idx(0)
    fire_gather(0)
    wait_gather(1)
    stage_idx(3, 1)
    wait_out()
    transpose_scale(1)
    fire_out(1, 1)

    @pl.loop(2, SB_PER_W - 2, step=2)
    def _(s0):
        for u in range(2):
            s = s0 + u
            b = u
            wait_idx(1 - b)
            fire_gather(1 - b)
            wait_gather(b)
            stage_idx(s + 2, b)
            wait_out()
            transpose_scale(b)
            fire_out(s, b)

    s2 = SB_PER_W - 2
    wait_idx(1)
    fire_gather(1)
    wait_gather(0)
    wait_out()
    transpose_scale(0)
    fire_out(s2, 0)

    wait_gather(1)
    wait_out()
    transpose_scale(1)
    fire_out(s2 + 1, 1)
    wait_out()


@jax.jit
def _emb(weight, idx2d):
    mesh = plsc.VectorSubcoreMesh(
        core_axis_name="c", subcore_axis_name="s", num_cores=NC, num_subcores=NS
    )
    run = pl.kernel(
        _emb_kernel,
        out_type=jax.ShapeDtypeStruct((NT, FR, N_BLK_B, 8, GRP), jnp.float32),
        mesh=mesh,
        scratch_types=[
            pltpu.VMEM((2, NDESC, DESC), jnp.int32),
            pltpu.VMEM((2, SB_ROWS, D), jnp.float32),
            pltpu.VMEM((FR, KB, 8, GRP + 1), jnp.float32),
            pltpu.SemaphoreType.DMA,
            pltpu.SemaphoreType.DMA,
            pltpu.SemaphoreType.DMA,
            pltpu.SemaphoreType.DMA,
        ],
        compiler_params=pltpu.CompilerParams(
            use_tc_tiling_on_sc=False, needs_layout_passes=False
        ),
    )
    return run(weight, idx2d)


def kernel(input_ids, weight):
    idx2d = input_ids.astype(jnp.int32).T.reshape(B_TOTAL // DESC, DESC)
    raw5 = _emb(weight, idx2d)
    out = raw5.transpose(2, 4, 0, 1, 3).reshape(NB, NT, D)
    return out
